# Initial kernel scaffold; baseline (speedup 1.0000x reference)
import jax
import jax.numpy as jnp
from jax.experimental import pallas as pl


def kernel(x, edge_index, weight, bias):
    raise NotImplementedError("write your pallas kernel here")


# speedup vs baseline: 20.3868x; 20.3868x over previous
import functools

import jax
import jax.numpy as jnp
from jax import lax
from jax.experimental import pallas as pl
from jax.experimental.pallas import tpu as pltpu
from jax.experimental.pallas import tpu_sc as plsc

NC = 2
NS = 16
NW = NC * NS
DEG_W = 128


def _sc_mesh():
    return plsc.VectorSubcoreMesh(
        core_axis_name="c", subcore_axis_name="s", num_cores=NC, num_subcores=NS
    )


def _degree_body(np_, nch, c, ei_hbm, ones_hbm, zeros_hbm, degw_hbm, idx_v, ones_v, acc_sh):
    cid = lax.axis_index("c")
    sid = lax.axis_index("s")
    wid = sid * NC + cid
    rps = np_ // NS

    pltpu.sync_copy(
        zeros_hbm.at[pl.ds(sid * rps, rps)], acc_sh.at[pl.ds(sid * rps, rps)]
    )
    pltpu.sync_copy(ones_hbm, ones_v)
    pltpu.sync_copy(ei_hbm.at[wid], idx_v)
    plsc.subcore_barrier()

    @pl.loop(0, nch)
    def _(j):
        pltpu.sync_copy(ones_v, acc_sh.at[idx_v.at[j]], add=True)

    plsc.subcore_barrier()
    pltpu.sync_copy(
        acc_sh.at[pl.ds(sid * rps, rps)],
        degw_hbm.at[cid, pl.ds(sid * rps, rps)],
    )


def _sc_degree(eir, ones_c, zeros_d):
    nw, nch, c = eir.shape
    np_ = zeros_d.shape[0]
    body = functools.partial(_degree_body, np_, nch, c)
    return pl.kernel(
        body,
        out_type=jax.ShapeDtypeStruct((NC, np_, DEG_W), jnp.float32),
        mesh=_sc_mesh(),
        scratch_types=[
            pltpu.VMEM((nch, c), jnp.int32),
            pltpu.VMEM((c, DEG_W), jnp.float32),
            pltpu.VMEM_SHARED((np_, DEG_W), jnp.float32),
        ],
    )(eir, ones_c, zeros_d)


def _agg_body(np_, nch, c, d, y_hbm, ei_hbm, ej_hbm, zeros_hbm, part_hbm,
              ei_v, ej_v, rows_v, acc_sh):
    cid = lax.axis_index("c")
    sid = lax.axis_index("s")
    wid = sid * NC + cid
    rps = np_ // NS

    pltpu.sync_copy(
        zeros_hbm.at[pl.ds(sid * rps, rps)], acc_sh.at[pl.ds(sid * rps, rps)]
    )
    pltpu.sync_copy(ei_hbm.at[wid], ei_v)
    pltpu.sync_copy(ej_hbm.at[wid], ej_v)
    plsc.subcore_barrier()

    @pl.loop(0, nch)
    def _(j):
        pltpu.sync_copy(y_hbm.at[ei_v.at[j]], rows_v)
        pltpu.sync_copy(rows_v, acc_sh.at[ej_v.at[j]], add=True)

    plsc.subcore_barrier()
    pltpu.sync_copy(
        acc_sh.at[pl.ds(sid * rps, rps)],
        part_hbm.at[cid, pl.ds(sid * rps, rps)],
    )


def _sc_aggregate(y, eir, ejr, zeros_y):
    nw, nch, c = eir.shape
    d = y.shape[1]
    np_ = zeros_y.shape[0]
    body = functools.partial(_agg_body, np_, nch, c, d)
    return pl.kernel(
        body,
        out_type=jax.ShapeDtypeStruct((NC, np_, d), jnp.float32),
        mesh=_sc_mesh(),
        scratch_types=[
            pltpu.VMEM((nch, c), jnp.int32),
            pltpu.VMEM((nch, c), jnp.int32),
            pltpu.VMEM((c, d), jnp.float32),
            pltpu.VMEM_SHARED((np_, d), jnp.float32),
        ],
    )(y, eir, ejr, zeros_y)


def _matmul_body(x_ref, w_ref, o_ref):
    o_ref[...] = jnp.dot(x_ref[...], w_ref[...], preferred_element_type=jnp.float32)


def _tc_matmul(x, w, bn=2000):
    n, k = x.shape
    d = w.shape[1]
    return pl.pallas_call(
        _matmul_body,
        grid=(n // bn,),
        in_specs=[
            pl.BlockSpec((bn, k), lambda i: (i, 0)),
            pl.BlockSpec((k, d), lambda i: (0, 0)),
        ],
        out_specs=pl.BlockSpec((bn, d), lambda i: (i, 0)),
        out_shape=jax.ShapeDtypeStruct((n, d), jnp.float32),
    )(x, w)


def _inv_sqrt_deg(degw_blk):
    deg = degw_blk[0, :, 0:1] + degw_blk[1, :, 0:1]
    return jnp.where(deg > 0.0, lax.rsqrt(deg), 0.0)


def _scale_body(degw_ref, xw_ref, o_ref):
    o_ref[...] = _inv_sqrt_deg(degw_ref[...]) * xw_ref[...]


def _tc_scale(degw, xw, bn=2000):
    n, d = xw.shape
    return pl.pallas_call(
        _scale_body,
        grid=(n // bn,),
        in_specs=[
            pl.BlockSpec((NC, bn, DEG_W), lambda i: (0, i, 0)),
            pl.BlockSpec((bn, d), lambda i: (i, 0)),
        ],
        out_specs=pl.BlockSpec((bn, d), lambda i: (i, 0)),
        out_shape=jax.ShapeDtypeStruct((n, d), jnp.float32),
    )(degw, xw)


def _final_body(degw_ref, part_ref, bias_ref, o_ref):
    s = _inv_sqrt_deg(degw_ref[...])
    acc = part_ref[0] + part_ref[1]
    o_ref[...] = s * acc + bias_ref[...]


def _tc_final(degw, parts, bias2d, n, bn=2000):
    d = parts.shape[2]
    return pl.pallas_call(
        _final_body,
        grid=(n // bn,),
        in_specs=[
            pl.BlockSpec((NC, bn, DEG_W), lambda i: (0, i, 0)),
            pl.BlockSpec((NC, bn, d), lambda i: (0, i, 0)),
            pl.BlockSpec((1, d), lambda i: (0, 0)),
        ],
        out_specs=pl.BlockSpec((bn, d), lambda i: (i, 0)),
        out_shape=jax.ShapeDtypeStruct((n, d), jnp.float32),
    )(degw, parts, bias2d)


def kernel(x, edge_index, weight, bias):
    n, _ = x.shape
    d = weight.shape[1]
    e = edge_index.shape[1]
    assert e % NW == 0 and n % NS == 0
    epw = e // NW
    c = 80
    assert epw % c == 0
    nch = epw // c

    npad = -(-n // (NS * 8)) * (NS * 8)

    eir = edge_index[0].reshape(NW, nch, c)
    ejr = edge_index[1].reshape(NW, nch, c)
    ones_c = jnp.ones((c, DEG_W), jnp.float32)
    zeros_nd = jnp.zeros((npad, d), jnp.float32)

    degw = _sc_degree(eir, ones_c, zeros_nd)
    xw = _tc_matmul(x, weight)
    y = _tc_scale(degw, xw)
    parts = _sc_aggregate(y, eir, ejr, zeros_nd)
    return _tc_final(degw, parts, bias.reshape(1, d), n)


# speedup vs baseline: 21.0189x; 1.0310x over previous
import functools

import jax
import jax.numpy as jnp
from jax import lax
from jax.experimental import pallas as pl
from jax.experimental.pallas import tpu as pltpu
from jax.experimental.pallas import tpu_sc as plsc

NC = 2
NS = 16
NW = NC * NS
DEG_W = 128


def _sc_mesh():
    return plsc.VectorSubcoreMesh(
        core_axis_name="c", subcore_axis_name="s", num_cores=NC, num_subcores=NS
    )


def _degree_body(np_, nmac, ms, c, ei_hbm, ones_hbm, zeros_hbm, degw_hbm,
                 idx_v, ones_v, sem, acc_sh):
    cid = lax.axis_index("c")
    sid = lax.axis_index("s")
    wid = sid * NC + cid
    rps = np_ // NS

    pltpu.sync_copy(
        zeros_hbm.at[pl.ds(sid * rps, rps)], acc_sh.at[pl.ds(sid * rps, rps)]
    )
    pltpu.sync_copy(ones_hbm, ones_v)
    pltpu.sync_copy(ei_hbm.at[wid], idx_v)
    plsc.subcore_barrier()

    g = 5
    @pl.loop(0, nmac)
    def _(m):
        @pl.loop(0, ms, step=g)
        def _(k):
            descs = [
                pltpu.async_copy(ones_v, acc_sh.at[idx_v.at[m, k + b]], sem, add=True)
                for b in range(g)
            ]
            for dsc in descs:
                dsc.wait()

    plsc.subcore_barrier()
    pltpu.sync_copy(
        acc_sh.at[pl.ds(sid * rps, rps)],
        degw_hbm.at[cid, pl.ds(sid * rps, rps)],
    )


def _sc_degree(eir, ones_c, zeros_d):
    nw, nmac, ms, c = eir.shape
    np_ = zeros_d.shape[0]
    body = functools.partial(_degree_body, np_, nmac, ms, c)
    return pl.kernel(
        body,
        out_type=jax.ShapeDtypeStruct((NC, np_, DEG_W), jnp.float32),
        mesh=_sc_mesh(),
        scratch_types=[
            pltpu.VMEM((nmac, ms, c), jnp.int32),
            pltpu.VMEM((c, DEG_W), jnp.float32),
            pltpu.SemaphoreType.DMA,
            pltpu.VMEM_SHARED((np_, DEG_W), jnp.float32),
        ],
    )(eir, ones_c, zeros_d)


def _agg_body(np_, nmac, ms, c, d, g, y_hbm, ei_hbm, ej_hbm, zeros_hbm, part_hbm,
              ei_v, ej_v, *scr):
    cid = lax.axis_index("c")
    sid = lax.axis_index("s")
    wid = sid * NC + cid
    rps = np_ // NS
    rows = scr[:g]
    ssem = scr[g]
    gsems = scr[g + 1 : 2 * g + 1]
    acc_sh = scr[2 * g + 1]

    pltpu.sync_copy(
        zeros_hbm.at[pl.ds(sid * rps, rps)], acc_sh.at[pl.ds(sid * rps, rps)]
    )
    plsc.subcore_barrier()

    @pl.loop(0, nmac)
    def _(m):
        pltpu.sync_copy(ei_hbm.at[wid, m], ei_v)
        pltpu.sync_copy(ej_hbm.at[wid, m], ej_v)

        @pl.loop(0, ms, step=g)
        def _(k):
            gds = [
                pltpu.async_copy(y_hbm.at[ei_v.at[k + b]], rows[b], gsems[b])
                for b in range(g)
            ]
            sds = []
            for b in range(g):
                gds[b].wait()
                sds.append(
                    pltpu.async_copy(rows[b], acc_sh.at[ej_v.at[k + b]], ssem, add=True)
                )
            for sd in sds:
                sd.wait()

    plsc.subcore_barrier()
    pltpu.sync_copy(
        acc_sh.at[pl.ds(sid * rps, rps)],
        part_hbm.at[cid, pl.ds(sid * rps, rps)],
    )


def _sc_aggregate(y, eir, ejr, zeros_y, g=2):
    nw, nmac, ms, c = eir.shape
    d = y.shape[1]
    np_ = zeros_y.shape[0]
    assert ms % g == 0
    body = functools.partial(_agg_body, np_, nmac, ms, c, d, g)
    return pl.kernel(
        body,
        out_type=jax.ShapeDtypeStruct((NC, np_, d), jnp.float32),
        mesh=_sc_mesh(),
        scratch_types=[
            pltpu.VMEM((ms, c), jnp.int32),
            pltpu.VMEM((ms, c), jnp.int32),
        ]
        + [pltpu.VMEM((c, d), jnp.float32)] * g
        + [pltpu.SemaphoreType.DMA]
        + [pltpu.SemaphoreType.DMA] * g
        + [pltpu.VMEM_SHARED((np_, d), jnp.float32)],
    )(y, eir, ejr, zeros_y)


def _matmul_body(x_ref, w_ref, o_ref):
    o_ref[...] = jnp.dot(x_ref[...], w_ref[...], preferred_element_type=jnp.float32)


def _tc_matmul(x, w, bn=2000):
    n, k = x.shape
    d = w.shape[1]
    return pl.pallas_call(
        _matmul_body,
        grid=(n // bn,),
        in_specs=[
            pl.BlockSpec((bn, k), lambda i: (i, 0)),
            pl.BlockSpec((k, d), lambda i: (0, 0)),
        ],
        out_specs=pl.BlockSpec((bn, d), lambda i: (i, 0)),
        out_shape=jax.ShapeDtypeStruct((n, d), jnp.float32),
    )(x, w)


def _inv_sqrt_deg(degw_blk):
    deg = degw_blk[0, :, 0:1] + degw_blk[1, :, 0:1]
    return jnp.where(deg > 0.0, lax.rsqrt(deg), 0.0)


def _scale_body(degw_ref, xw_ref, o_ref):
    o_ref[...] = _inv_sqrt_deg(degw_ref[...]) * xw_ref[...]


def _tc_scale(degw, xw, bn=2000):
    n, d = xw.shape
    return pl.pallas_call(
        _scale_body,
        grid=(n // bn,),
        in_specs=[
            pl.BlockSpec((NC, bn, DEG_W), lambda i: (0, i, 0)),
            pl.BlockSpec((bn, d), lambda i: (i, 0)),
        ],
        out_specs=pl.BlockSpec((bn, d), lambda i: (i, 0)),
        out_shape=jax.ShapeDtypeStruct((n, d), jnp.float32),
    )(degw, xw)


def _final_body(degw_ref, part_ref, bias_ref, o_ref):
    s = _inv_sqrt_deg(degw_ref[...])
    acc = part_ref[0] + part_ref[1]
    o_ref[...] = s * acc + bias_ref[...]


def _tc_final(degw, parts, bias2d, n, bn=2000):
    d = parts.shape[2]
    return pl.pallas_call(
        _final_body,
        grid=(n // bn,),
        in_specs=[
            pl.BlockSpec((NC, bn, DEG_W), lambda i: (0, i, 0)),
            pl.BlockSpec((NC, bn, d), lambda i: (0, i, 0)),
            pl.BlockSpec((1, d), lambda i: (0, 0)),
        ],
        out_specs=pl.BlockSpec((bn, d), lambda i: (i, 0)),
        out_shape=jax.ShapeDtypeStruct((n, d), jnp.float32),
    )(degw, parts, bias2d)


def kernel(x, edge_index, weight, bias):
    n, _ = x.shape
    d = weight.shape[1]
    e = edge_index.shape[1]
    assert e % NW == 0 and n % NS == 0
    epw = e // NW
    c = 40
    ms = 50
    assert epw % (c * ms) == 0
    nmac = epw // (c * ms)

    npad = -(-n // (NS * 8)) * (NS * 8)

    eir = edge_index[0].reshape(NW, nmac, ms, c)
    ejr = edge_index[1].reshape(NW, nmac, ms, c)
    ones_c = jnp.ones((c, DEG_W), jnp.float32)
    zeros_nd = jnp.zeros((npad, d), jnp.float32)

    degw = _sc_degree(eir, ones_c, zeros_nd)
    xw = _tc_matmul(x, weight)
    y = _tc_scale(degw, xw)
    parts = _sc_aggregate(y, eir, ejr, zeros_nd)
    return _tc_final(degw, parts, bias.reshape(1, d), n)


# speedup vs baseline: 24.4941x; 1.1653x over previous
import functools

import jax
import jax.numpy as jnp
from jax import lax
from jax.experimental import pallas as pl
from jax.experimental.pallas import tpu as pltpu
from jax.experimental.pallas import tpu_sc as plsc

NC = 2
NS = 16
NW = NC * NS
DEG_W = 128


def _sc_mesh():
    return plsc.VectorSubcoreMesh(
        core_axis_name="c", subcore_axis_name="s", num_cores=NC, num_subcores=NS
    )


def _degree_body(np_, nch, c, ei_hbm, ones_hbm, zeros_hbm, degw_hbm,
                 idx_v, ones_v, sem, acc_sh):
    cid = lax.axis_index("c")
    sid = lax.axis_index("s")
    wid = sid * NC + cid
    rps = np_ // NS

    pltpu.sync_copy(
        zeros_hbm.at[pl.ds(sid * rps, rps)], acc_sh.at[pl.ds(sid * rps, rps)]
    )
    pltpu.sync_copy(ones_hbm, ones_v)
    pltpu.sync_copy(ei_hbm.at[wid], idx_v)
    plsc.subcore_barrier()

    g = 5
    @pl.loop(0, nch, step=g)
    def _(k):
        descs = [
            pltpu.async_copy(ones_v, acc_sh.at[idx_v.at[k + b]], sem, add=True)
            for b in range(g)
        ]
        for dsc in descs:
            dsc.wait()

    plsc.subcore_barrier()
    pltpu.sync_copy(
        acc_sh.at[pl.ds(sid * rps, rps)],
        degw_hbm.at[cid, pl.ds(sid * rps, rps)],
    )


def _sc_degree(eir, ones_c, zeros_d):
    nw, nch, c = eir.shape
    np_ = zeros_d.shape[0]
    body = functools.partial(_degree_body, np_, nch, c)
    return pl.kernel(
        body,
        out_type=jax.ShapeDtypeStruct((NC, np_, DEG_W), jnp.float32),
        mesh=_sc_mesh(),
        scratch_types=[
            pltpu.VMEM((nch, c), jnp.int32),
            pltpu.VMEM((c, DEG_W), jnp.float32),
            pltpu.SemaphoreType.DMA,
            pltpu.VMEM_SHARED((np_, DEG_W), jnp.float32),
        ],
    )(eir, ones_c, zeros_d)


def _agg_body(np_, nmac, ms, c, d, g, y_hbm, ei_hbm, ej_hbm, zeros_hbm, part_hbm,
              *scr):
    cid = lax.axis_index("c")
    sid = lax.axis_index("s")
    wid = sid * NC + cid
    rps = np_ // NS
    ei_v = scr[0:2]
    ej_v = scr[2:4]
    rows = scr[4 : 4 + g]
    isems = scr[4 + g : 6 + g]
    ssem = scr[6 + g]
    gsems = scr[7 + g : 7 + 2 * g]
    acc_sh = scr[7 + 2 * g]

    pltpu.sync_copy(
        zeros_hbm.at[pl.ds(sid * rps, rps)], acc_sh.at[pl.ds(sid * rps, rps)]
    )
    pltpu.sync_copy(ei_hbm.at[wid, 0], ei_v[0])
    pltpu.sync_copy(ej_hbm.at[wid, 0], ej_v[0])
    plsc.subcore_barrier()

    def load_idx(m, p):
        return (
            pltpu.async_copy(ei_hbm.at[wid, m], ei_v[p], isems[0]),
            pltpu.async_copy(ej_hbm.at[wid, m], ej_v[p], isems[1]),
        )

    def group(k, p):
        gds = [
            pltpu.async_copy(y_hbm.at[ei_v[p].at[k + b]], rows[b], gsems[b])
            for b in range(g)
        ]
        sds = []
        for b in range(g):
            gds[b].wait()
            sds.append(
                pltpu.async_copy(rows[b], acc_sh.at[ej_v[p].at[k + b]], ssem, add=True)
            )
        for sd in sds:
            sd.wait()

    def macro(m, p, last):
        if not last:
            nxt = load_idx(m + 1, 1 - p)

        @pl.loop(0, ms, step=g)
        def _(k):
            group(k, p)

        if not last:
            nxt[0].wait()
            nxt[1].wait()

    @pl.loop(0, nmac - 2, step=2)
    def _(mm):
        macro(mm, 0, False)
        macro(mm + 1, 1, False)

    macro(nmac - 2, 0, False)
    macro(nmac - 1, 1, True)

    plsc.subcore_barrier()
    pltpu.sync_copy(
        acc_sh.at[pl.ds(sid * rps, rps)],
        part_hbm.at[cid, pl.ds(sid * rps, rps)],
    )


def _sc_aggregate(y, eir, ejr, zeros_y, g=5):
    nw, nmac, ms, c = eir.shape
    d = y.shape[1]
    np_ = zeros_y.shape[0]
    assert ms % g == 0 and nmac % 2 == 0 and nmac >= 4
    body = functools.partial(_agg_body, np_, nmac, ms, c, d, g)
    return pl.kernel(
        body,
        out_type=jax.ShapeDtypeStruct((NC, np_, d), jnp.float32),
        mesh=_sc_mesh(),
        scratch_types=[pltpu.VMEM((ms, c), jnp.int32)] * 4
        + [pltpu.VMEM((c, d), jnp.float32)] * g
        + [pltpu.SemaphoreType.DMA] * 2
        + [pltpu.SemaphoreType.DMA]
        + [pltpu.SemaphoreType.DMA] * g
        + [pltpu.VMEM_SHARED((np_, d), jnp.float32)],
    )(y, eir, ejr, zeros_y)


def _matmul_body(x_ref, w_ref, o_ref):
    o_ref[...] = jnp.dot(x_ref[...], w_ref[...], preferred_element_type=jnp.float32)


def _tc_matmul(x, w, bn=2000):
    n, k = x.shape
    d = w.shape[1]
    return pl.pallas_call(
        _matmul_body,
        grid=(n // bn,),
        in_specs=[
            pl.BlockSpec((bn, k), lambda i: (i, 0)),
            pl.BlockSpec((k, d), lambda i: (0, 0)),
        ],
        out_specs=pl.BlockSpec((bn, d), lambda i: (i, 0)),
        out_shape=jax.ShapeDtypeStruct((n, d), jnp.float32),
    )(x, w)


def _inv_sqrt_deg(degw_blk):
    deg = degw_blk[0, :, 0:1] + degw_blk[1, :, 0:1]
    return jnp.where(deg > 0.0, lax.rsqrt(deg), 0.0)


def _scale_body(degw_ref, xw_ref, o_ref):
    o_ref[...] = _inv_sqrt_deg(degw_ref[...]) * xw_ref[...]


def _tc_scale(degw, xw, bn=2000):
    n, d = xw.shape
    return pl.pallas_call(
        _scale_body,
        grid=(n // bn,),
        in_specs=[
            pl.BlockSpec((NC, bn, DEG_W), lambda i: (0, i, 0)),
            pl.BlockSpec((bn, d), lambda i: (i, 0)),
        ],
        out_specs=pl.BlockSpec((bn, d), lambda i: (i, 0)),
        out_shape=jax.ShapeDtypeStruct((n, d), jnp.float32),
    )(degw, xw)


def _final_body(degw_ref, part_ref, bias_ref, o_ref):
    s = _inv_sqrt_deg(degw_ref[...])
    acc = part_ref[0] + part_ref[1]
    o_ref[...] = s * acc + bias_ref[...]


def _tc_final(degw, parts, bias2d, n, bn=2000):
    d = parts.shape[2]
    return pl.pallas_call(
        _final_body,
        grid=(n // bn,),
        in_specs=[
            pl.BlockSpec((NC, bn, DEG_W), lambda i: (0, i, 0)),
            pl.BlockSpec((NC, bn, d), lambda i: (0, i, 0)),
            pl.BlockSpec((1, d), lambda i: (0, 0)),
        ],
        out_specs=pl.BlockSpec((bn, d), lambda i: (i, 0)),
        out_shape=jax.ShapeDtypeStruct((n, d), jnp.float32),
    )(degw, parts, bias2d)


def kernel(x, edge_index, weight, bias):
    n, _ = x.shape
    d = weight.shape[1]
    e = edge_index.shape[1]
    assert e % NW == 0 and n % NS == 0
    epw = e // NW
    c = 40
    ms = 25
    assert epw % (c * ms) == 0
    nch = epw // c
    nmac = nch // ms

    npad = -(-n // (NS * 8)) * (NS * 8)

    eir = edge_index[0].reshape(NW, nch, c)
    ejr = edge_index[1].reshape(NW, nch, c)
    eir4 = eir.reshape(NW, nmac, ms, c)
    ejr4 = ejr.reshape(NW, nmac, ms, c)
    ones_c = jnp.ones((c, DEG_W), jnp.float32)
    zeros_nd = jnp.zeros((npad, d), jnp.float32)

    degw = _sc_degree(eir, ones_c, zeros_nd)
    xw = _tc_matmul(x, weight)
    y = _tc_scale(degw, xw)
    parts = _sc_aggregate(y, eir4, ejr4, zeros_nd)
    return _tc_final(degw, parts, bias.reshape(1, d), n)


# speedup vs baseline: 24.5351x; 1.0017x over previous
import functools

import jax
import jax.numpy as jnp
from jax import lax
from jax.experimental import pallas as pl
from jax.experimental.pallas import tpu as pltpu
from jax.experimental.pallas import tpu_sc as plsc

NC = 2
NS = 16
NW = NC * NS
DEG_W = 128


def _sc_mesh():
    return plsc.VectorSubcoreMesh(
        core_axis_name="c", subcore_axis_name="s", num_cores=NC, num_subcores=NS
    )


def _degree_body(np_, nch, c, ei_hbm, ones_hbm, zeros_hbm, degw_hbm,
                 idx_v, ones_v, sem, acc_sh):
    cid = lax.axis_index("c")
    sid = lax.axis_index("s")
    wid = sid * NC + cid
    rps = np_ // NS

    pltpu.sync_copy(
        zeros_hbm.at[pl.ds(sid * rps, rps)], acc_sh.at[pl.ds(sid * rps, rps)]
    )
    pltpu.sync_copy(ones_hbm, ones_v)
    pltpu.sync_copy(ei_hbm.at[wid], idx_v)
    plsc.subcore_barrier()

    g = 5
    @pl.loop(0, nch, step=g)
    def _(k):
        descs = [
            pltpu.async_copy(ones_v, acc_sh.at[idx_v.at[k + b]], sem, add=True)
            for b in range(g)
        ]
        for dsc in descs:
            dsc.wait()

    plsc.subcore_barrier()
    pltpu.sync_copy(
        acc_sh.at[pl.ds(sid * rps, rps)],
        degw_hbm.at[cid, pl.ds(sid * rps, rps)],
    )


def _sc_degree(eir, ones_c, zeros_d):
    nw, nch, c = eir.shape
    np_ = zeros_d.shape[0]
    body = functools.partial(_degree_body, np_, nch, c)
    return pl.kernel(
        body,
        out_type=jax.ShapeDtypeStruct((NC, np_, DEG_W), jnp.float32),
        mesh=_sc_mesh(),
        scratch_types=[
            pltpu.VMEM((nch, c), jnp.int32),
            pltpu.VMEM((c, DEG_W), jnp.float32),
            pltpu.SemaphoreType.DMA,
            pltpu.VMEM_SHARED((np_, DEG_W), jnp.float32),
        ],
    )(eir, ones_c, zeros_d)


def _agg_body(np_, nmac, ms, c, d, g, y_hbm, ei_hbm, ej_hbm, zeros_hbm, part_hbm,
              *scr):
    cid = lax.axis_index("c")
    sid = lax.axis_index("s")
    wid = sid * NC + cid
    rps = np_ // NS
    ei_v = scr[0:2]
    ej_v = scr[2:4]
    rows = scr[4 : 4 + g]
    isems = scr[4 + g : 6 + g]
    ssem = scr[6 + g]
    gsems = scr[7 + g : 7 + 2 * g]
    acc_sh = scr[7 + 2 * g]

    pltpu.sync_copy(
        zeros_hbm.at[pl.ds(sid * rps, rps)], acc_sh.at[pl.ds(sid * rps, rps)]
    )
    pltpu.sync_copy(ei_hbm.at[wid, 0], ei_v[0])
    pltpu.sync_copy(ej_hbm.at[wid, 0], ej_v[0])
    plsc.subcore_barrier()

    def load_idx(m, p):
        return (
            pltpu.async_copy(ei_hbm.at[wid, m], ei_v[p], isems[0]),
            pltpu.async_copy(ej_hbm.at[wid, m], ej_v[p], isems[1]),
        )

    def group(k, p):
        gds = [
            pltpu.async_copy(y_hbm.at[ei_v[p].at[k + b]], rows[b], gsems[b])
            for b in range(g)
        ]
        sds = []
        for b in range(g):
            gds[b].wait()
            sds.append(
                pltpu.async_copy(rows[b], acc_sh.at[ej_v[p].at[k + b]], ssem, add=True)
            )
        for sd in sds:
            sd.wait()

    def macro(m, p, last):
        if not last:
            nxt = load_idx(m + 1, 1 - p)

        @pl.loop(0, ms, step=g)
        def _(k):
            group(k, p)

        if not last:
            nxt[0].wait()
            nxt[1].wait()

    @pl.loop(0, nmac - 2, step=2)
    def _(mm):
        macro(mm, 0, False)
        macro(mm + 1, 1, False)

    macro(nmac - 2, 0, False)
    macro(nmac - 1, 1, True)

    plsc.subcore_barrier()
    pltpu.sync_copy(
        acc_sh.at[pl.ds(sid * rps, rps)],
        part_hbm.at[cid, pl.ds(sid * rps, rps)],
    )


def _sc_aggregate(y, eir, ejr, zeros_y, g=5):
    nw, nmac, ms, c = eir.shape
    d = y.shape[1]
    np_ = zeros_y.shape[0]
    assert ms % g == 0 and nmac % 2 == 0 and nmac >= 4
    body = functools.partial(_agg_body, np_, nmac, ms, c, d, g)
    return pl.kernel(
        body,
        out_type=jax.ShapeDtypeStruct((NC, np_, d), jnp.float32),
        mesh=_sc_mesh(),
        scratch_types=[pltpu.VMEM((ms, c), jnp.int32)] * 4
        + [pltpu.VMEM((c, d), jnp.float32)] * g
        + [pltpu.SemaphoreType.DMA] * 2
        + [pltpu.SemaphoreType.DMA]
        + [pltpu.SemaphoreType.DMA] * g
        + [pltpu.VMEM_SHARED((np_, d), jnp.float32)],
    )(y, eir, ejr, zeros_y)


def _matmul_body(x_ref, w_ref, o_ref):
    o_ref[...] = jnp.dot(x_ref[...], w_ref[...], preferred_element_type=jnp.float32)


def _tc_matmul(x, w, bn=2000):
    n, k = x.shape
    d = w.shape[1]
    return pl.pallas_call(
        _matmul_body,
        grid=(n // bn,),
        in_specs=[
            pl.BlockSpec((bn, k), lambda i: (i, 0)),
            pl.BlockSpec((k, d), lambda i: (0, 0)),
        ],
        out_specs=pl.BlockSpec((bn, d), lambda i: (i, 0)),
        out_shape=jax.ShapeDtypeStruct((n, d), jnp.float32),
    )(x, w)


def _inv_sqrt_deg(degw_blk):
    deg = degw_blk[0, :, 0:1] + degw_blk[1, :, 0:1]
    return jnp.where(deg > 0.0, lax.rsqrt(deg), 0.0)


def _scale_body(degw_ref, xw_ref, o_ref):
    o_ref[...] = _inv_sqrt_deg(degw_ref[...]) * xw_ref[...]


def _tc_scale(degw, xw, bn=2000):
    n, d = xw.shape
    return pl.pallas_call(
        _scale_body,
        grid=(n // bn,),
        in_specs=[
            pl.BlockSpec((NC, bn, 1), lambda i: (0, i, 0)),
            pl.BlockSpec((bn, d), lambda i: (i, 0)),
        ],
        out_specs=pl.BlockSpec((bn, d), lambda i: (i, 0)),
        out_shape=jax.ShapeDtypeStruct((n, d), jnp.float32),
    )(degw, xw)


def _final_body(degw_ref, part_ref, bias_ref, o_ref):
    s = _inv_sqrt_deg(degw_ref[...])
    acc = part_ref[0] + part_ref[1]
    o_ref[...] = s * acc + bias_ref[...]


def _tc_final(degw, parts, bias2d, n, bn=2000):
    d = parts.shape[2]
    return pl.pallas_call(
        _final_body,
        grid=(n // bn,),
        in_specs=[
            pl.BlockSpec((NC, bn, 1), lambda i: (0, i, 0)),
            pl.BlockSpec((NC, bn, d), lambda i: (0, i, 0)),
            pl.BlockSpec((1, d), lambda i: (0, 0)),
        ],
        out_specs=pl.BlockSpec((bn, d), lambda i: (i, 0)),
        out_shape=jax.ShapeDtypeStruct((n, d), jnp.float32),
    )(degw, parts, bias2d)


def kernel(x, edge_index, weight, bias):
    n, _ = x.shape
    d = weight.shape[1]
    e = edge_index.shape[1]
    assert e % NW == 0 and n % NS == 0
    epw = e // NW
    c = 40
    ms = 25
    assert epw % (c * ms) == 0
    nch = epw // c
    nmac = nch // ms

    npad = -(-n // (NS * 8)) * (NS * 8)

    eir = edge_index[0].reshape(NW, nch, c)
    ejr = edge_index[1].reshape(NW, nch, c)
    eir4 = eir.reshape(NW, nmac, ms, c)
    ejr4 = ejr.reshape(NW, nmac, ms, c)
    ones_c = jnp.ones((c, DEG_W), jnp.float32)
    zeros_nd = jnp.zeros((npad, d), jnp.float32)

    degw = _sc_degree(eir, ones_c, zeros_nd)
    degc = degw[:, :, :1]
    xw = _tc_matmul(x, weight)
    y = _tc_scale(degc, xw)
    parts = _sc_aggregate(y, eir4, ejr4, zeros_nd)
    return _tc_final(degc, parts, bias.reshape(1, d), n)


# speedup vs baseline: 31.2054x; 1.2719x over previous
import dataclasses
import functools

import jax
import jax.numpy as jnp
from jax import lax
from jax.experimental import pallas as pl
from jax.experimental.pallas import tpu as pltpu
from jax.experimental.pallas import tpu_sc as plsc

NC = 2
NS = 16
NW = NC * NS
DEG_W = 128


def _sc_mesh():
    return plsc.VectorSubcoreMesh(
        core_axis_name="c", subcore_axis_name="s", num_cores=NC, num_subcores=NS
    )


def _degree_body(np_, epw, ei_hbm, degc_hbm, idx_v, hist_v, col_v, res_v, grid_sh):
    cid = lax.axis_index("c")
    sid = lax.axis_index("s")
    wid = sid * NC + cid
    rps = np_ // NS

    z16 = jnp.zeros((16,), jnp.float32)

    @pl.loop(0, np_, step=16)
    def _(i):
        hist_v[pl.ds(i, 16)] = z16

    pltpu.sync_copy(ei_hbm.at[wid], idx_v)

    ones16 = jnp.ones((16,), jnp.float32)

    @pl.loop(0, epw, step=16)
    def _(i):
        plsc.addupdate_scatter(hist_v, [idx_v[pl.ds(i, 16)]], ones16)

    for t in range(NS):
        pltpu.sync_copy(hist_v.at[pl.ds(t * rps, rps)], grid_sh.at[t, sid])
    plsc.subcore_barrier()
    pltpu.sync_copy(grid_sh.at[sid], col_v)

    @pl.loop(0, rps, step=16)
    def _(k):
        acc = col_v[0, pl.ds(k, 16)]
        for t in range(1, NS):
            acc = acc + col_v[t, pl.ds(k, 16)]
        res_v[pl.ds(k, 16)] = acc

    pltpu.sync_copy(res_v, degc_hbm.at[cid, 0, pl.ds(sid * rps, rps)])


def _sc_degree(ei2, np_):
    nw, epw = ei2.shape
    cp = pltpu.CompilerParams()
    if "needs_layout_passes" in pltpu.CompilerParams.__dataclass_fields__:
        cp = dataclasses.replace(cp, needs_layout_passes=False)
    body = functools.partial(_degree_body, np_, epw)
    return pl.kernel(
        body,
        out_type=jax.ShapeDtypeStruct((NC, 1, np_), jnp.float32),
        mesh=_sc_mesh(),
        compiler_params=cp,
        scratch_types=[
            pltpu.VMEM((epw,), jnp.int32),
            pltpu.VMEM((np_,), jnp.float32),
            pltpu.VMEM((NS, np_ // NS), jnp.float32),
            pltpu.VMEM((np_ // NS,), jnp.float32),
            pltpu.VMEM_SHARED((NS, NS, np_ // NS), jnp.float32),
        ],
    )(ei2)


def _agg_body(np_, nmac, ms, c, d, g, y_hbm, ei_hbm, ej_hbm, zeros_hbm, part_hbm,
              *scr):
    cid = lax.axis_index("c")
    sid = lax.axis_index("s")
    wid = sid * NC + cid
    rps = np_ // NS
    ei_v = scr[0:2]
    ej_v = scr[2:4]
    rows = scr[4 : 4 + g]
    isems = scr[4 + g : 6 + g]
    ssem = scr[6 + g]
    gsems = scr[7 + g : 7 + 2 * g]
    acc_sh = scr[7 + 2 * g]

    pltpu.sync_copy(
        zeros_hbm.at[pl.ds(sid * rps, rps)], acc_sh.at[pl.ds(sid * rps, rps)]
    )
    pltpu.sync_copy(ei_hbm.at[wid, 0], ei_v[0])
    pltpu.sync_copy(ej_hbm.at[wid, 0], ej_v[0])
    plsc.subcore_barrier()

    def load_idx(m, p):
        return (
            pltpu.async_copy(ei_hbm.at[wid, m], ei_v[p], isems[0]),
            pltpu.async_copy(ej_hbm.at[wid, m], ej_v[p], isems[1]),
        )

    def group(k, p):
        gds = [
            pltpu.async_copy(y_hbm.at[ei_v[p].at[k + b]], rows[b], gsems[b])
            for b in range(g)
        ]
        sds = []
        for b in range(g):
            gds[b].wait()
            sds.append(
                pltpu.async_copy(rows[b], acc_sh.at[ej_v[p].at[k + b]], ssem, add=True)
            )
        for sd in sds:
            sd.wait()

    def macro(m, p, last):
        if not last:
            nxt = load_idx(m + 1, 1 - p)

        @pl.loop(0, ms, step=g)
        def _(k):
            group(k, p)

        if not last:
            nxt[0].wait()
            nxt[1].wait()

    @pl.loop(0, nmac - 2, step=2)
    def _(mm):
        macro(mm, 0, False)
        macro(mm + 1, 1, False)

    macro(nmac - 2, 0, False)
    macro(nmac - 1, 1, True)

    plsc.subcore_barrier()
    pltpu.sync_copy(
        acc_sh.at[pl.ds(sid * rps, rps)],
        part_hbm.at[cid, pl.ds(sid * rps, rps)],
    )


def _sc_aggregate(y, eir, ejr, zeros_y, g=5):
    nw, nmac, ms, c = eir.shape
    d = y.shape[1]
    np_ = zeros_y.shape[0]
    assert ms % g == 0 and nmac % 2 == 0 and nmac >= 4
    body = functools.partial(_agg_body, np_, nmac, ms, c, d, g)
    return pl.kernel(
        body,
        out_type=jax.ShapeDtypeStruct((NC, np_, d), jnp.float32),
        mesh=_sc_mesh(),
        scratch_types=[pltpu.VMEM((ms, c), jnp.int32)] * 4
        + [pltpu.VMEM((c, d), jnp.float32)] * g
        + [pltpu.SemaphoreType.DMA] * 2
        + [pltpu.SemaphoreType.DMA]
        + [pltpu.SemaphoreType.DMA] * g
        + [pltpu.VMEM_SHARED((np_, d), jnp.float32)],
    )(y, eir, ejr, zeros_y)


def _matmul_body(x_ref, w_ref, o_ref):
    o_ref[...] = jnp.dot(x_ref[...], w_ref[...], preferred_element_type=jnp.float32)


def _tc_matmul(x, w, bn=2000):
    n, k = x.shape
    d = w.shape[1]
    return pl.pallas_call(
        _matmul_body,
        grid=(n // bn,),
        in_specs=[
            pl.BlockSpec((bn, k), lambda i: (i, 0)),
            pl.BlockSpec((k, d), lambda i: (0, 0)),
        ],
        out_specs=pl.BlockSpec((bn, d), lambda i: (i, 0)),
        out_shape=jax.ShapeDtypeStruct((n, d), jnp.float32),
    )(x, w)


def _inv_sqrt_deg(degt_blk):
    deg = degt_blk[:, 0:1] + degt_blk[:, 1:2]
    return jnp.where(deg > 0.0, lax.rsqrt(deg), 0.0)


def _scale_body(degt_ref, xw_ref, o_ref):
    o_ref[...] = _inv_sqrt_deg(degt_ref[...]) * xw_ref[...]


def _tc_scale(degt, xw, bn=2000):
    n, d = xw.shape
    return pl.pallas_call(
        _scale_body,
        grid=(n // bn,),
        in_specs=[
            pl.BlockSpec((bn, NC), lambda i: (i, 0)),
            pl.BlockSpec((bn, d), lambda i: (i, 0)),
        ],
        out_specs=pl.BlockSpec((bn, d), lambda i: (i, 0)),
        out_shape=jax.ShapeDtypeStruct((n, d), jnp.float32),
    )(degt, xw)


def _final_body(degt_ref, part_ref, bias_ref, o_ref):
    s = _inv_sqrt_deg(degt_ref[...])
    acc = part_ref[0] + part_ref[1]
    o_ref[...] = s * acc + bias_ref[...]


def _tc_final(degt, parts, bias2d, n, bn=2000):
    d = parts.shape[2]
    return pl.pallas_call(
        _final_body,
        grid=(n // bn,),
        in_specs=[
            pl.BlockSpec((bn, NC), lambda i: (i, 0)),
            pl.BlockSpec((NC, bn, d), lambda i: (0, i, 0)),
            pl.BlockSpec((1, d), lambda i: (0, 0)),
        ],
        out_specs=pl.BlockSpec((bn, d), lambda i: (i, 0)),
        out_shape=jax.ShapeDtypeStruct((n, d), jnp.float32),
    )(degt, parts, bias2d)


def kernel(x, edge_index, weight, bias):
    n, _ = x.shape
    d = weight.shape[1]
    e = edge_index.shape[1]
    assert e % NW == 0 and n % NS == 0
    epw = e // NW
    c = 40
    ms = 25
    assert epw % (c * ms) == 0
    nch = epw // c
    nmac = nch // ms

    npad = -(-n // (NS * 16)) * (NS * 16)

    ei2 = edge_index[0].reshape(NW, epw)
    eir4 = edge_index[0].reshape(NW, nmac, ms, c)
    ejr4 = edge_index[1].reshape(NW, nmac, ms, c)
    zeros_nd = jnp.zeros((npad, d), jnp.float32)

    degc = _sc_degree(ei2, npad)
    degt = degc.reshape(NC, npad).T
    xw = _tc_matmul(x, weight)
    y = _tc_scale(degt, xw)
    parts = _sc_aggregate(y, eir4, ejr4, zeros_nd)
    return _tc_final(degt, parts, bias.reshape(1, d), n)


# speedup vs baseline: 31.5177x; 1.0100x over previous
import dataclasses
import functools

import jax
import jax.numpy as jnp
from jax import lax
from jax.experimental import pallas as pl
from jax.experimental.pallas import tpu as pltpu
from jax.experimental.pallas import tpu_sc as plsc

NC = 2
NS = 16
NW = NC * NS
DEG_W = 128


def _sc_mesh():
    return plsc.VectorSubcoreMesh(
        core_axis_name="c", subcore_axis_name="s", num_cores=NC, num_subcores=NS
    )


def _degree_body(np_, epw, ei_hbm, degc_hbm, idx_v, hist_v, col_v, res_v, grid_sh):
    cid = lax.axis_index("c")
    sid = lax.axis_index("s")
    wid = sid * NC + cid
    rps = np_ // NS

    z16 = jnp.zeros((16,), jnp.float32)

    @pl.loop(0, np_, step=16)
    def _(i):
        hist_v[pl.ds(i, 16)] = z16

    pltpu.sync_copy(ei_hbm.at[wid], idx_v)

    ones16 = jnp.ones((16,), jnp.float32)

    @pl.loop(0, epw, step=16)
    def _(i):
        plsc.addupdate_scatter(hist_v, [idx_v[pl.ds(i, 16)]], ones16)

    for t in range(NS):
        pltpu.sync_copy(hist_v.at[pl.ds(t * rps, rps)], grid_sh.at[t, sid])
    plsc.subcore_barrier()
    pltpu.sync_copy(grid_sh.at[sid], col_v)

    @pl.loop(0, rps, step=16)
    def _(k):
        acc = col_v[0, pl.ds(k, 16)]
        for t in range(1, NS):
            acc = acc + col_v[t, pl.ds(k, 16)]
        res_v[pl.ds(k, 16)] = acc

    pltpu.sync_copy(res_v, degc_hbm.at[cid, 0, pl.ds(sid * rps, rps)])


def _sc_degree(ei2, np_):
    nw, epw = ei2.shape
    cp = pltpu.CompilerParams()
    if "needs_layout_passes" in pltpu.CompilerParams.__dataclass_fields__:
        cp = dataclasses.replace(cp, needs_layout_passes=False)
    body = functools.partial(_degree_body, np_, epw)
    return pl.kernel(
        body,
        out_type=jax.ShapeDtypeStruct((NC, 1, np_), jnp.float32),
        mesh=_sc_mesh(),
        compiler_params=cp,
        scratch_types=[
            pltpu.VMEM((epw,), jnp.int32),
            pltpu.VMEM((np_,), jnp.float32),
            pltpu.VMEM((NS, np_ // NS), jnp.float32),
            pltpu.VMEM((np_ // NS,), jnp.float32),
            pltpu.VMEM_SHARED((NS, NS, np_ // NS), jnp.float32),
        ],
    )(ei2)


def _agg_body(np_, nmac, ms, c, d, g, y_hbm, ei_hbm, ej_hbm, zeros_hbm, part_hbm,
              *scr):
    cid = lax.axis_index("c")
    sid = lax.axis_index("s")
    wid = sid * NC + cid
    rps = np_ // NS
    ei_v = scr[0:2]
    ej_v = scr[2:4]
    rows = scr[4 : 4 + g]
    isems = scr[4 + g : 6 + g]
    ssem = scr[6 + g]
    gsems = scr[7 + g : 7 + 2 * g]
    acc_sh = scr[7 + 2 * g]

    pltpu.sync_copy(zeros_hbm, acc_sh.at[pl.ds(sid * rps, rps)])
    pltpu.sync_copy(ei_hbm.at[wid, 0], ei_v[0])
    pltpu.sync_copy(ej_hbm.at[wid, 0], ej_v[0])
    plsc.subcore_barrier()

    def load_idx(m, p):
        return (
            pltpu.async_copy(ei_hbm.at[wid, m], ei_v[p], isems[0]),
            pltpu.async_copy(ej_hbm.at[wid, m], ej_v[p], isems[1]),
        )

    def group(k, p):
        gds = [
            pltpu.async_copy(y_hbm.at[ei_v[p].at[k + b]], rows[b], gsems[b])
            for b in range(g)
        ]
        sds = []
        for b in range(g):
            gds[b].wait()
            sds.append(
                pltpu.async_copy(rows[b], acc_sh.at[ej_v[p].at[k + b]], ssem, add=True)
            )
        for sd in sds:
            sd.wait()

    def macro(m, p, last):
        if not last:
            nxt = load_idx(m + 1, 1 - p)

        @pl.loop(0, ms, step=g)
        def _(k):
            group(k, p)

        if not last:
            nxt[0].wait()
            nxt[1].wait()

    @pl.loop(0, nmac - 2, step=2)
    def _(mm):
        macro(mm, 0, False)
        macro(mm + 1, 1, False)

    macro(nmac - 2, 0, False)
    macro(nmac - 1, 1, True)

    plsc.subcore_barrier()
    pltpu.sync_copy(
        acc_sh.at[pl.ds(sid * rps, rps)],
        part_hbm.at[cid, pl.ds(sid * rps, rps)],
    )


def _sc_aggregate(y, eir, ejr, zeros_y, np_, g=5):
    nw, nmac, ms, c = eir.shape
    d = y.shape[1]
    assert ms % g == 0 and nmac % 2 == 0 and nmac >= 4
    body = functools.partial(_agg_body, np_, nmac, ms, c, d, g)
    return pl.kernel(
        body,
        out_type=jax.ShapeDtypeStruct((NC, np_, d), jnp.float32),
        mesh=_sc_mesh(),
        scratch_types=[pltpu.VMEM((ms, c), jnp.int32)] * 4
        + [pltpu.VMEM((c, d), jnp.float32)] * g
        + [pltpu.SemaphoreType.DMA] * 2
        + [pltpu.SemaphoreType.DMA]
        + [pltpu.SemaphoreType.DMA] * g
        + [pltpu.VMEM_SHARED((np_, d), jnp.float32)],
    )(y, eir, ejr, zeros_y)


def _inv_sqrt_deg(degt_blk):
    deg = degt_blk[:, 0:1] + degt_blk[:, 1:2]
    return jnp.where(deg > 0.0, lax.rsqrt(deg), 0.0)


def _matmul_scale_body(degt_ref, x_ref, w_ref, o_ref):
    xw = jnp.dot(x_ref[...], w_ref[...], preferred_element_type=jnp.float32)
    o_ref[...] = _inv_sqrt_deg(degt_ref[...]) * xw


def _tc_matmul_scale(degt, x, w, bn=2000):
    n, k = x.shape
    d = w.shape[1]
    return pl.pallas_call(
        _matmul_scale_body,
        grid=(n // bn,),
        in_specs=[
            pl.BlockSpec((bn, NC), lambda i: (i, 0)),
            pl.BlockSpec((bn, k), lambda i: (i, 0)),
            pl.BlockSpec((k, d), lambda i: (0, 0)),
        ],
        out_specs=pl.BlockSpec((bn, d), lambda i: (i, 0)),
        out_shape=jax.ShapeDtypeStruct((n, d), jnp.float32),
    )(degt, x, w)


def _final_body(degt_ref, part_ref, bias_ref, o_ref):
    s = _inv_sqrt_deg(degt_ref[...])
    acc = part_ref[0] + part_ref[1]
    o_ref[...] = s * acc + bias_ref[...]


def _tc_final(degt, parts, bias2d, n, bn=2000):
    d = parts.shape[2]
    return pl.pallas_call(
        _final_body,
        grid=(n // bn,),
        in_specs=[
            pl.BlockSpec((bn, NC), lambda i: (i, 0)),
            pl.BlockSpec((NC, bn, d), lambda i: (0, i, 0)),
            pl.BlockSpec((1, d), lambda i: (0, 0)),
        ],
        out_specs=pl.BlockSpec((bn, d), lambda i: (i, 0)),
        out_shape=jax.ShapeDtypeStruct((n, d), jnp.float32),
    )(degt, parts, bias2d)


def kernel(x, edge_index, weight, bias):
    n, _ = x.shape
    d = weight.shape[1]
    e = edge_index.shape[1]
    assert e % NW == 0 and n % NS == 0
    epw = e // NW
    c = 40
    ms = 25
    assert epw % (c * ms) == 0
    nch = epw // c
    nmac = nch // ms

    npad = -(-n // (NS * 16)) * (NS * 16)

    ei2 = edge_index[0].reshape(NW, epw)
    eir4 = edge_index[0].reshape(NW, nmac, ms, c)
    ejr4 = edge_index[1].reshape(NW, nmac, ms, c)
    zeros_rd = jnp.zeros((npad // NS, d), jnp.float32)

    degc = _sc_degree(ei2, npad)
    degt = degc.reshape(NC, npad).T
    y = _tc_matmul_scale(degt, x, weight)
    parts = _sc_aggregate(y, eir4, ejr4, zeros_rd, npad)
    return _tc_final(degt, parts, bias.reshape(1, d), n)


# speedup vs baseline: 31.6928x; 1.0056x over previous
import dataclasses
import functools

import jax
import jax.numpy as jnp
from jax import lax
from jax.experimental import pallas as pl
from jax.experimental.pallas import tpu as pltpu
from jax.experimental.pallas import tpu_sc as plsc

NC = 2
NS = 16
NW = NC * NS
DEG_W = 128


def _sc_mesh():
    return plsc.VectorSubcoreMesh(
        core_axis_name="c", subcore_axis_name="s", num_cores=NC, num_subcores=NS
    )


def _degree_body(np_, epw, ei_hbm, degc_hbm, idx_v, hist_v, col_v, res_v, grid_sh):
    cid = lax.axis_index("c")
    sid = lax.axis_index("s")
    wid = sid * NC + cid
    rps = np_ // NS

    z16 = jnp.zeros((16,), jnp.float32)

    @pl.loop(0, np_, step=16)
    def _(i):
        hist_v[pl.ds(i, 16)] = z16

    pltpu.sync_copy(ei_hbm.at[wid], idx_v)

    ones16 = jnp.ones((16,), jnp.float32)

    @pl.loop(0, epw, step=16)
    def _(i):
        plsc.addupdate_scatter(hist_v, [idx_v[pl.ds(i, 16)]], ones16)

    for t in range(NS):
        pltpu.sync_copy(hist_v.at[pl.ds(t * rps, rps)], grid_sh.at[t, sid])
    plsc.subcore_barrier()
    pltpu.sync_copy(grid_sh.at[sid], col_v)

    @pl.loop(0, rps, step=16)
    def _(k):
        acc = col_v[0, pl.ds(k, 16)]
        for t in range(1, NS):
            acc = acc + col_v[t, pl.ds(k, 16)]
        res_v[pl.ds(k, 16)] = acc

    pltpu.sync_copy(res_v, degc_hbm.at[cid, 0, pl.ds(sid * rps, rps)])


def _sc_degree(ei2, np_):
    nw, epw = ei2.shape
    cp = pltpu.CompilerParams()
    if "needs_layout_passes" in pltpu.CompilerParams.__dataclass_fields__:
        cp = dataclasses.replace(cp, needs_layout_passes=False)
    body = functools.partial(_degree_body, np_, epw)
    return pl.kernel(
        body,
        out_type=jax.ShapeDtypeStruct((NC, 1, np_), jnp.float32),
        mesh=_sc_mesh(),
        compiler_params=cp,
        scratch_types=[
            pltpu.VMEM((epw,), jnp.int32),
            pltpu.VMEM((np_,), jnp.float32),
            pltpu.VMEM((NS, np_ // NS), jnp.float32),
            pltpu.VMEM((np_ // NS,), jnp.float32),
            pltpu.VMEM_SHARED((NS, NS, np_ // NS), jnp.float32),
        ],
    )(ei2)


def _agg_body(np_, nmac, ms, c, d, g, y_hbm, ei_hbm, ej_hbm, zeros_hbm, part_hbm,
              *scr):
    cid = lax.axis_index("c")
    sid = lax.axis_index("s")
    wid = sid * NC + cid
    rps = np_ // NS
    ei_v = scr[0:2]
    ej_v = scr[2:4]
    rows = scr[4 : 4 + 2 * g]
    isems = scr[4 + 2 * g : 6 + 2 * g]
    ssem = scr[6 + 2 * g]
    gsems = scr[7 + 2 * g : 7 + 4 * g]
    acc_sh = scr[7 + 4 * g]

    pltpu.sync_copy(zeros_hbm, acc_sh.at[pl.ds(sid * rps, rps)])
    pltpu.sync_copy(ei_hbm.at[wid, 0], ei_v[0])
    pltpu.sync_copy(ej_hbm.at[wid, 0], ej_v[0])
    plsc.subcore_barrier()

    def load_idx(m, p):
        return (
            pltpu.async_copy(ei_hbm.at[wid, m], ei_v[p], isems[0]),
            pltpu.async_copy(ej_hbm.at[wid, m], ej_v[p], isems[1]),
        )

    h = g

    def fire_gathers(k, p, base):
        return [
            pltpu.async_copy(
                y_hbm.at[ei_v[p].at[k + b]], rows[base + b], gsems[base + b]
            )
            for b in range(h)
        ]

    def scatter_half(k, p, base):
        sds = []
        for b in range(h):
            sds.append(
                pltpu.async_copy(
                    rows[base + b], acc_sh.at[ej_v[p].at[k + b]], ssem, add=True
                )
            )
        return sds

    def wait_gather_a(k, p):
        for b in range(h):
            pltpu.make_async_copy(
                y_hbm.at[ei_v[p].at[k + b]], rows[b], gsems[b]
            ).wait()

    def macro(m, p, last):
        if not last:
            nxt = load_idx(m + 1, 1 - p)

        main = ((ms - 2 * h) // (2 * h)) * (2 * h)
        fire_gathers(0, p, 0)

        @pl.loop(0, main, step=2 * h)
        def _(k):
            bds = fire_gathers(k + h, p, h)
            wait_gather_a(k, p)
            sa = scatter_half(k, p, 0)
            for sd in sa:
                sd.wait()

            @pl.when(k + 2 * h < main)
            def _():
                fire_gathers(k + 2 * h, p, 0)

            for bd in bds:
                bd.wait()
            sb = scatter_half(k + h, p, h)
            for sd in sb:
                sd.wait()

        k = main
        while k < ms:
            nb = min(h, ms - k)
            gds = [
                pltpu.async_copy(
                    y_hbm.at[ei_v[p].at[k + b]], rows[b], gsems[b]
                )
                for b in range(nb)
            ]
            sds = []
            for b in range(nb):
                gds[b].wait()
                sds.append(
                    pltpu.async_copy(
                        rows[b], acc_sh.at[ej_v[p].at[k + b]], ssem, add=True
                    )
                )
            for sd in sds:
                sd.wait()
            k += nb

        if not last:
            nxt[0].wait()
            nxt[1].wait()

    if nmac % 2 == 0:
        @pl.loop(0, nmac - 2, step=2)
        def _(mm):
            macro(mm, 0, False)
            macro(mm + 1, 1, False)

        macro(nmac - 2, 0, False)
        macro(nmac - 1, 1, True)
    else:
        @pl.loop(0, nmac - 1, step=2)
        def _(mm):
            macro(mm, 0, False)
            macro(mm + 1, 1, False)

        macro(nmac - 1, 0, True)

    plsc.subcore_barrier()
    pltpu.sync_copy(
        acc_sh.at[pl.ds(sid * rps, rps)],
        part_hbm.at[cid, pl.ds(sid * rps, rps)],
    )


def _sc_aggregate(y, eir, ejr, zeros_y, np_, g=3):
    nw, nmac, ms, c = eir.shape
    d = y.shape[1]
    assert ms >= 2 * g and nmac >= 3
    body = functools.partial(_agg_body, np_, nmac, ms, c, d, g)
    return pl.kernel(
        body,
        out_type=jax.ShapeDtypeStruct((NC, np_, d), jnp.float32),
        mesh=_sc_mesh(),
        scratch_types=[pltpu.VMEM((ms, c), jnp.int32)] * 4
        + [pltpu.VMEM((c, d), jnp.float32)] * (2 * g)
        + [pltpu.SemaphoreType.DMA] * 2
        + [pltpu.SemaphoreType.DMA]
        + [pltpu.SemaphoreType.DMA] * (2 * g)
        + [pltpu.VMEM_SHARED((np_, d), jnp.float32)],
    )(y, eir, ejr, zeros_y)


def _inv_sqrt_deg(degt_blk):
    deg = degt_blk[:, 0:1] + degt_blk[:, 1:2]
    return jnp.where(deg > 0.0, lax.rsqrt(deg), 0.0)


def _matmul_scale_body(degt_ref, x_ref, w_ref, o_ref):
    xw = jnp.dot(x_ref[...], w_ref[...], preferred_element_type=jnp.float32)
    o_ref[...] = _inv_sqrt_deg(degt_ref[...]) * xw


def _tc_matmul_scale(degt, x, w, bn=2000):
    n, k = x.shape
    d = w.shape[1]
    return pl.pallas_call(
        _matmul_scale_body,
        grid=(n // bn,),
        in_specs=[
            pl.BlockSpec((bn, NC), lambda i: (i, 0)),
            pl.BlockSpec((bn, k), lambda i: (i, 0)),
            pl.BlockSpec((k, d), lambda i: (0, 0)),
        ],
        out_specs=pl.BlockSpec((bn, d), lambda i: (i, 0)),
        out_shape=jax.ShapeDtypeStruct((n, d), jnp.float32),
    )(degt, x, w)


def _final_body(degt_ref, part_ref, bias_ref, o_ref):
    s = _inv_sqrt_deg(degt_ref[...])
    acc = part_ref[0] + part_ref[1]
    o_ref[...] = s * acc + bias_ref[...]


def _tc_final(degt, parts, bias2d, n, bn=2000):
    d = parts.shape[2]
    return pl.pallas_call(
        _final_body,
        grid=(n // bn,),
        in_specs=[
            pl.BlockSpec((bn, NC), lambda i: (i, 0)),
            pl.BlockSpec((NC, bn, d), lambda i: (0, i, 0)),
            pl.BlockSpec((1, d), lambda i: (0, 0)),
        ],
        out_specs=pl.BlockSpec((bn, d), lambda i: (i, 0)),
        out_shape=jax.ShapeDtypeStruct((n, d), jnp.float32),
    )(degt, parts, bias2d)


def kernel(x, edge_index, weight, bias):
    n, _ = x.shape
    d = weight.shape[1]
    e = edge_index.shape[1]
    assert e % NW == 0 and n % NS == 0
    epw = e // NW
    c = 40
    ms = 25
    assert epw % (c * ms) == 0
    nch = epw // c
    nmac = nch // ms

    npad = -(-n // (NS * 16)) * (NS * 16)

    ei2 = edge_index[0].reshape(NW, epw)
    eir4 = edge_index[0].reshape(NW, nmac, ms, c)
    ejr4 = edge_index[1].reshape(NW, nmac, ms, c)
    zeros_rd = jnp.zeros((npad // NS, d), jnp.float32)

    degc = _sc_degree(ei2, npad)
    degt = degc.reshape(NC, npad).T
    y = _tc_matmul_scale(degt, x, weight)
    parts = _sc_aggregate(y, eir4, ejr4, zeros_rd, npad)
    return _tc_final(degt, parts, bias.reshape(1, d), n)
